# baseline (speedup 1.0000x reference)
import functools

import jax
import jax.numpy as jnp
from jax import lax
from jax.experimental import pallas as pl
from jax.experimental.pallas import tpu as pltpu
from jax.experimental.pallas import tpu_sc as plsc

N = 50000
C = 128
KVOL = 27
E = 23000
NT = 16
KP = 9
NPH = KVOL // KP
PE = KP * E
BLK = 128
NBLK = 104
EPAD = NT * NBLK * BLK
Q = 32
NP = 50048
RPT = NP // NT
ECH = 136
NCH = RPT // ECH
DUMMY = NP
ACC_ROWS = NP + 8
SB = 13
NCHUNK = NBLK // SB


def _gemm(x, w, BN):
    R = x.shape[0]

    def body(x_ref, w_ref, y_ref):
        y_ref[...] = lax.dot_general(
            x_ref[...], w_ref[0],
            (((1,), (1,)), ((), ())),
            preferred_element_type=jnp.float32)[None]

    return pl.pallas_call(
        body,
        grid=(R // BN, KP),
        in_specs=[
            pl.BlockSpec((BN, C), lambda i, k: (i, 0)),
            pl.BlockSpec((1, C, C), lambda i, k: (k, 0, 0)),
        ],
        out_specs=pl.BlockSpec((1, BN, C), lambda i, k: (k, i, 0)),
        out_shape=jax.ShapeDtypeStruct((KP, R, C), jnp.float32),
    )(x, w)


def _make_scatter(load_init, fin):
    mesh = plsc.VectorSubcoreMesh(core_axis_name="c", subcore_axis_name="s")
    second = fin == "ep2"

    scratch = [
        pltpu.VMEM_SHARED((ACC_ROWS, Q), jnp.float32),
        pltpu.VMEM((2, SB, BLK), jnp.int32),
        pltpu.VMEM((2, SB, BLK), jnp.int32),
        pltpu.VMEM((2, BLK, Q), jnp.float32),
        pltpu.VMEM((ECH, Q), jnp.float32),
        pltpu.VMEM((ECH, Q), jnp.float32),
        pltpu.VMEM((Q,), jnp.float32),
        pltpu.VMEM((Q,), jnp.float32),
        pltpu.VMEM((16,), jnp.float32),
        pltpu.SemaphoreType.DMA,
        pltpu.SemaphoreType.DMA,
        pltpu.SemaphoreType.DMA,
        pltpu.SemaphoreType.DMA,
        pltpu.SemaphoreType.DMA,
        pltpu.SemaphoreType.DMA,
    ]

    def body(y4_hbm, gidx_hbm, oidx_hbm, b_hbm, rq_hbm, slope_hbm, x_hbm,
             pin_hbm, out_hbm, acc, gidx_v, oidx_v, msg_v, ebuf, xbuf,
             b_v, rq_v, slope_v, sem_i0, sem_i1, sem_g0, sem_g1, sem_s0,
             sem_s1):
        c = lax.axis_index("c")
        s = lax.axis_index("s")
        sem_i = [sem_i0, sem_i1]
        sem_g = [sem_g0, sem_g1]
        sem_s = [sem_s0, sem_s1]
        zeros16 = jnp.zeros((16,), jnp.float32)

        def idx_start(q, cc, t):
            pltpu.async_copy(gidx_hbm.at[q, s, pl.ds(cc * SB, SB)],
                             gidx_v.at[t], sem_i[t])
            pltpu.async_copy(oidx_hbm.at[s, pl.ds(cc * SB, SB)],
                             oidx_v.at[t], sem_i[t])

        def idx_wait(q, cc, t):
            pltpu.make_async_copy(gidx_hbm.at[q, s, pl.ds(cc * SB, SB)],
                                  gidx_v.at[t], sem_i[t]).wait()
            pltpu.make_async_copy(oidx_hbm.at[s, pl.ds(cc * SB, SB)],
                                  oidx_v.at[t], sem_i[t]).wait()

        def do_chunk(q, cc, t):
            idx_wait(q, cc, t)
            gv, ov = gidx_v.at[t], oidx_v.at[t]
            pltpu.async_copy(y4_hbm.at[gv.at[0]], msg_v.at[0], sem_g[0])
            for j in range(SB):
                b, nb = j & 1, (j + 1) & 1
                if j + 1 < SB:
                    if j >= 1:
                        pltpu.make_async_copy(
                            msg_v.at[nb], acc.at[ov.at[j - 1]],
                            sem_s[nb]).wait()
                    pltpu.async_copy(y4_hbm.at[gv.at[j + 1]],
                                     msg_v.at[nb], sem_g[nb])
                pltpu.make_async_copy(y4_hbm.at[gv.at[j]], msg_v.at[b],
                                      sem_g[b]).wait()
                pltpu.async_copy(msg_v.at[b], acc.at[ov.at[j]], sem_s[b],
                                 add=True)
            pltpu.make_async_copy(msg_v.at[(SB - 2) & 1],
                                  acc.at[ov.at[SB - 2]],
                                  sem_s[(SB - 2) & 1]).wait()
            pltpu.make_async_copy(msg_v.at[(SB - 1) & 1],
                                  acc.at[ov.at[SB - 1]],
                                  sem_s[(SB - 1) & 1]).wait()

        for p in range(2):
            q = 2 * c + p
            qoff = pl.multiple_of(q * Q, Q)

            if load_init:
                def lcopy(z, _):
                    r0 = pl.multiple_of(s * RPT + z * ECH, 8)
                    pltpu.sync_copy(
                        pin_hbm.at[pl.ds(r0, ECH), pl.ds(qoff, Q)], ebuf)
                    pltpu.sync_copy(ebuf, acc.at[pl.ds(r0, ECH)])
                    return 0
                lax.fori_loop(0, NCH, lcopy, 0)
            else:
                def zbody(i, _):
                    ebuf[i, pl.ds(0, 16)] = zeros16
                    ebuf[i, pl.ds(16, 16)] = zeros16
                    return 0
                lax.fori_loop(0, ECH, zbody, 0)

                def zcopy(z, _):
                    r0 = pl.multiple_of(s * RPT + z * ECH, 8)
                    pltpu.sync_copy(ebuf, acc.at[pl.ds(r0, ECH)])
                    return 0
                lax.fori_loop(0, NCH, zcopy, 0)
            plsc.subcore_barrier()

            idx_start(q, 0, 0)
            idx_start(q, 1, 1)

            def cbody(c2, _):
                for t in range(2):
                    cc = 2 * c2 + t
                    do_chunk(q, cc, t)

                    @pl.when(cc + 2 < NCHUNK)
                    def _():
                        idx_start(q, cc + 2, t)
                return 0
            lax.fori_loop(0, NCHUNK // 2, cbody, 0)
            plsc.subcore_barrier()

            if fin != "dump":
                pltpu.sync_copy(b_hbm.at[pl.ds(qoff, Q)], b_v)
                pltpu.sync_copy(rq_hbm.at[pl.ds(qoff, Q)], rq_v)
                pltpu.sync_copy(slope_hbm, slope_v)
                slope = slope_v[...]
            for e in range(NCH):
                r0 = pl.multiple_of(s * RPT + e * ECH, 8)
                pltpu.sync_copy(acc.at[pl.ds(r0, ECH)], ebuf)
                if fin != "dump":
                    if second:
                        pltpu.sync_copy(
                            x_hbm.at[pl.ds(r0, ECH), pl.ds(qoff, Q)],
                            xbuf)

                    def ebody(r, _):
                        for hh in range(2):
                            sl = pl.ds(hh * 16, 16)
                            v = ebuf[r, sl] + b_v[sl]
                            if second:
                                v = v * rq_v[sl] + xbuf[r, sl]
                                v = jnp.where(v >= 0, v, v * slope)
                            else:
                                v = jnp.where(v >= 0, v, v * slope)
                                v = v * rq_v[sl]
                            ebuf[r, sl] = v
                        return 0
                    lax.fori_loop(0, ECH, ebody, 0)
                pltpu.sync_copy(
                    ebuf, out_hbm.at[pl.ds(r0, ECH), pl.ds(qoff, Q)])
            plsc.subcore_barrier()

    return functools.partial(
        pl.kernel, mesh=mesh,
        compiler_params=pltpu.CompilerParams(use_tc_tiling_on_sc=False),
        out_type=jax.ShapeDtypeStruct((NP, C), jnp.float32),
        scratch_types=scratch)(body)


_sc_zd = _make_scatter(False, "dump")
_sc_ld = _make_scatter(True, "dump")
_sc_ep1 = _make_scatter(True, "ep1")
_sc_ep2 = _make_scatter(True, "ep2")


def _make_epilogue(second):
    mesh = plsc.VectorSubcoreMesh(core_axis_name="c", subcore_axis_name="s")

    def body(pin_hbm, b_hbm, rq_hbm, slope_hbm, x_hbm, out_hbm, ebuf,
             xbuf, b_v, rq_v, slope_v):
        c = lax.axis_index("c")
        s = lax.axis_index("s")
        for p in range(2):
            q = 2 * c + p
            qoff = pl.multiple_of(q * Q, Q)
            pltpu.sync_copy(b_hbm.at[pl.ds(qoff, Q)], b_v)
            pltpu.sync_copy(rq_hbm.at[pl.ds(qoff, Q)], rq_v)
            pltpu.sync_copy(slope_hbm, slope_v)
            slope = slope_v[...]
            for e in range(NCH):
                r0 = pl.multiple_of(s * RPT + e * ECH, 8)
                pltpu.sync_copy(
                    pin_hbm.at[pl.ds(r0, ECH), pl.ds(qoff, Q)], ebuf)
                if second:
                    pltpu.sync_copy(
                        x_hbm.at[pl.ds(r0, ECH), pl.ds(qoff, Q)], xbuf)

                def ebody(r, _):
                    for hh in range(2):
                        sl = pl.ds(hh * 16, 16)
                        v = ebuf[r, sl] + b_v[sl]
                        if second:
                            v = v * rq_v[sl] + xbuf[r, sl]
                            v = jnp.where(v >= 0, v, v * slope)
                        else:
                            v = jnp.where(v >= 0, v, v * slope)
                            v = v * rq_v[sl]
                        ebuf[r, sl] = v
                    return 0
                lax.fori_loop(0, ECH, ebody, 0)
                pltpu.sync_copy(
                    ebuf, out_hbm.at[pl.ds(r0, ECH), pl.ds(qoff, Q)])

    return functools.partial(
        pl.kernel, mesh=mesh,
        compiler_params=pltpu.CompilerParams(use_tc_tiling_on_sc=False),
        out_type=jax.ShapeDtypeStruct((NP, C), jnp.float32),
        scratch_types=[
            pltpu.VMEM((ECH, Q), jnp.float32),
            pltpu.VMEM((ECH, Q), jnp.float32),
            pltpu.VMEM((Q,), jnp.float32),
            pltpu.VMEM((Q,), jnp.float32),
            pltpu.VMEM((16,), jnp.float32),
        ])(body)


_sc_epo1 = _make_epilogue(False)
_sc_epo2 = _make_epilogue(True)


def kernel(in_feats, w1, b1, slope1, rq1, w2, b2, rq2, slope2, in_maps,
           out_maps):
    x = in_feats

    def mk_g4(rows, ph):
        koff = jnp.arange(KP, dtype=jnp.int32)[:, None] * rows
        g = ((koff + in_maps[ph * KP:(ph + 1) * KP]) * 4).reshape(-1)
        g = jnp.concatenate([g, jnp.zeros((EPAD - PE,), jnp.int32)])
        return (g[None, :] + jnp.arange(4, dtype=jnp.int32)[:, None]
                ).reshape(4, NT, NBLK, BLK)

    def mk_o(ph):
        o = out_maps[ph * KP:(ph + 1) * KP].reshape(-1)
        return jnp.concatenate(
            [o, jnp.full((EPAD - PE,), DUMMY, jnp.int32)]).reshape(
                NT, NBLK, BLK)

    os_ = [mk_o(p) for p in range(NPH)]
    s1 = jnp.broadcast_to(slope1, (16,))
    s2 = jnp.broadcast_to(slope2, (16,))
    xp = jnp.pad(x, ((0, NP - N), (0, 0)))
    dummy_pin = xp

    def conv(xin, w, bias, rq, slope, fin, BN):
        rows = xin.shape[0]
        g4 = [mk_g4(rows, p) for p in range(NPH)]
        ys = [_gemm(xin, w[p * KP:(p + 1) * KP], BN).reshape(-1, Q)
              for p in range(NPH)]
        p0 = _sc_zd(ys[0], g4[0], os_[0], bias, rq, slope, xp, dummy_pin)
        p01 = _sc_ld(ys[1], g4[1], os_[1], bias, rq, slope, xp, p0)
        acc = _sc_ld(ys[2], g4[2], os_[2], bias, rq, slope, xp, p01)
        epo = _sc_epo2 if fin == "ep2" else _sc_epo1
        return epo(acc, bias, rq, slope, xp)

    hp = conv(x, w1, b1, rq1, s1, "ep1", 2000)
    return conv(hp, w2, b2, rq2, s2, "ep2", 6256)[:N]


# speedup vs baseline: 1.6988x; 1.6988x over previous
import functools

import jax
import jax.numpy as jnp
from jax import lax
from jax.experimental import pallas as pl
from jax.experimental.pallas import tpu as pltpu
from jax.experimental.pallas import tpu_sc as plsc

N = 50000
C = 128
KVOL = 27
E = 23000
NT = 16
NE = KVOL * E
BLK = 128
NBLK = 304
EPAD = NT * NBLK * BLK
Q = 32
NP = 50048
RPT = NP // NT
ECH = 136
NCH = RPT // ECH
DUMMY = NP
ACC_ROWS = NP + 8
SB = 19
NCHUNK = NBLK // SB


def _gemm(x, w, BN):
    R = x.shape[0]

    def body(x_ref, w_ref, y_ref):
        y_ref[...] = lax.dot_general(
            x_ref[...], w_ref[0],
            (((1,), (1,)), ((), ())),
            preferred_element_type=jnp.float32)[None]

    return pl.pallas_call(
        body,
        grid=(R // BN, KVOL),
        in_specs=[
            pl.BlockSpec((BN, C), lambda i, k: (i, 0)),
            pl.BlockSpec((1, C, C), lambda i, k: (k, 0, 0)),
        ],
        out_specs=pl.BlockSpec((1, BN, C), lambda i, k: (k, i, 0)),
        out_shape=jax.ShapeDtypeStruct((KVOL, R, C), jnp.float32),
    )(x, w)


def _make_scatter(second):
    mesh = plsc.VectorSubcoreMesh(core_axis_name="c", subcore_axis_name="s")

    @functools.partial(
        pl.kernel, mesh=mesh,
        compiler_params=pltpu.CompilerParams(use_tc_tiling_on_sc=False),
        out_type=jax.ShapeDtypeStruct((NP, C), jnp.float32),
        scratch_types=[
            pltpu.VMEM_SHARED((ACC_ROWS, Q), jnp.float32),
            pltpu.VMEM((SB, BLK), jnp.int32),
            pltpu.VMEM((SB, BLK), jnp.int32),
            pltpu.VMEM((3, BLK, Q), jnp.float32),
            pltpu.VMEM((ECH, Q), jnp.float32),
            pltpu.VMEM((ECH, Q), jnp.float32),
            pltpu.VMEM((Q,), jnp.float32),
            pltpu.VMEM((Q,), jnp.float32),
            pltpu.VMEM((16,), jnp.float32),
            pltpu.SemaphoreType.DMA,
            pltpu.SemaphoreType.DMA,
            pltpu.SemaphoreType.DMA,
            pltpu.SemaphoreType.DMA,
            pltpu.SemaphoreType.DMA,
            pltpu.SemaphoreType.DMA,
        ])
    def k(y4_hbm, gidx_hbm, oidx_hbm, b_hbm, rq_hbm, slope_hbm, x_hbm,
          out_hbm, acc, gidx_v, oidx_v, msg_v, ebuf, xbuf, b_v, rq_v,
          slope_v, sem_g0, sem_g1, sem_g2, sem_s0, sem_s1, sem_s2):
        c = lax.axis_index("c")
        s = lax.axis_index("s")
        sem_g = [sem_g0, sem_g1, sem_g2]
        sem_s = [sem_s0, sem_s1, sem_s2]
        zeros16 = jnp.zeros((16,), jnp.float32)

        def do_chunk(q, cc):
            pltpu.sync_copy(gidx_hbm.at[q, s, pl.ds(cc * SB, SB)], gidx_v)
            pltpu.sync_copy(oidx_hbm.at[s, pl.ds(cc * SB, SB)], oidx_v)
            gv, ov = gidx_v, oidx_v
            pltpu.async_copy(y4_hbm.at[gv.at[0]], msg_v.at[0], sem_g[0])
            pltpu.async_copy(y4_hbm.at[gv.at[1]], msg_v.at[1], sem_g[1])
            for j in range(SB):
                a = j % 3
                pltpu.make_async_copy(y4_hbm.at[gv.at[j]], msg_v.at[a],
                                      sem_g[a]).wait()
                pltpu.async_copy(msg_v.at[a], acc.at[ov.at[j]], sem_s[a],
                                 add=True)
                if j + 2 < SB:
                    na = (j + 2) % 3
                    if j >= 1:
                        pltpu.make_async_copy(
                            msg_v.at[na], acc.at[ov.at[j - 1]],
                            sem_s[na]).wait()
                    pltpu.async_copy(y4_hbm.at[gv.at[j + 2]],
                                     msg_v.at[na], sem_g[na])
            for j in (SB - 3, SB - 2, SB - 1):
                pltpu.make_async_copy(msg_v.at[j % 3],
                                      acc.at[ov.at[j]],
                                      sem_s[j % 3]).wait()

        for p in range(2):
            q = 2 * c + p

            def zbody(i, _):
                ebuf[i, pl.ds(0, 16)] = zeros16
                ebuf[i, pl.ds(16, 16)] = zeros16
                return 0
            lax.fori_loop(0, ECH, zbody, 0)

            def zcopy(z, _):
                r0 = pl.multiple_of(s * RPT + z * ECH, 8)
                pltpu.sync_copy(ebuf, acc.at[pl.ds(r0, ECH)])
                return 0
            lax.fori_loop(0, NCH, zcopy, 0)
            plsc.subcore_barrier()

            def cbody(cc, _):
                do_chunk(q, cc)
                return 0
            lax.fori_loop(0, NCHUNK, cbody, 0)
            plsc.subcore_barrier()

            qoff = pl.multiple_of(q * Q, Q)
            pltpu.sync_copy(b_hbm.at[pl.ds(qoff, Q)], b_v)
            pltpu.sync_copy(rq_hbm.at[pl.ds(qoff, Q)], rq_v)
            pltpu.sync_copy(slope_hbm, slope_v)
            slope = slope_v[...]
            for e in range(NCH):
                r0 = pl.multiple_of(s * RPT + e * ECH, 8)
                pltpu.sync_copy(acc.at[pl.ds(r0, ECH)], ebuf)
                if second:
                    pltpu.sync_copy(
                        x_hbm.at[pl.ds(r0, ECH), pl.ds(qoff, Q)], xbuf)

                def ebody(r, _):
                    for hh in range(2):
                        sl = pl.ds(hh * 16, 16)
                        v = ebuf[r, sl] + b_v[sl]
                        if second:
                            v = v * rq_v[sl] + xbuf[r, sl]
                            v = jnp.where(v >= 0, v, v * slope)
                        else:
                            v = jnp.where(v >= 0, v, v * slope)
                            v = v * rq_v[sl]
                        ebuf[r, sl] = v
                    return 0
                lax.fori_loop(0, ECH, ebody, 0)
                pltpu.sync_copy(
                    ebuf, out_hbm.at[pl.ds(r0, ECH), pl.ds(qoff, Q)])
            plsc.subcore_barrier()

    return k


_scatter1 = _make_scatter(second=False)
_scatter2 = _make_scatter(second=True)


def kernel(in_feats, w1, b1, slope1, rq1, w2, b2, rq2, slope2, in_maps,
           out_maps):
    x = in_feats

    def mk_g4(rows):
        koff = jnp.arange(KVOL, dtype=jnp.int32)[:, None] * rows
        g = ((koff + in_maps) * 4).reshape(-1)
        g = jnp.concatenate([g, jnp.zeros((EPAD - NE,), jnp.int32)])
        return (g[None, :] + jnp.arange(4, dtype=jnp.int32)[:, None]
                ).reshape(4, NT, NBLK, BLK)

    g4a = mk_g4(N)
    g4b = mk_g4(NP)
    o = jnp.concatenate(
        [out_maps.reshape(-1),
         jnp.full((EPAD - NE,), DUMMY, jnp.int32)]).reshape(NT, NBLK, BLK)
    s1 = jnp.broadcast_to(slope1, (16,))
    s2 = jnp.broadcast_to(slope2, (16,))
    xp = jnp.pad(x, ((0, NP - N), (0, 0)))

    y1 = _gemm(x, w1, 2000).reshape(KVOL * N * 4, Q)
    hp = _scatter1(y1, g4a, o, b1, rq1, s1, xp)
    y2 = _gemm(hp, w2, 6256).reshape(KVOL * NP * 4, Q)
    return _scatter2(y2, g4b, o, b2, rq2, s2, xp)[:N]


# speedup vs baseline: 1.7887x; 1.0529x over previous
import functools

import jax
import jax.numpy as jnp
from jax import lax
from jax.experimental import pallas as pl
from jax.experimental.pallas import tpu as pltpu
from jax.experimental.pallas import tpu_sc as plsc

N = 50000
C = 128
KVOL = 27
E = 23000
NT = 16
NE = KVOL * E
BLK = 128
NBLK = 304
EPAD = NT * NBLK * BLK
Q = 32
NP = 50048
RPT = NP // NT
ECH = 136
NCH = RPT // ECH
DUMMY = NP
ACC_ROWS = NP + 8
SB = 19
NCHUNK = NBLK // SB


def _gemm(x, w, BN):
    R = x.shape[0]

    def body(x_ref, w_ref, y_ref):
        y_ref[...] = lax.dot_general(
            x_ref[...], w_ref[0],
            (((1,), (1,)), ((), ())),
            preferred_element_type=jnp.float32)[None]

    return pl.pallas_call(
        body,
        grid=(R // BN, KVOL),
        in_specs=[
            pl.BlockSpec((BN, C), lambda i, k: (i, 0)),
            pl.BlockSpec((1, C, C), lambda i, k: (k, 0, 0)),
        ],
        out_specs=pl.BlockSpec((1, BN, C), lambda i, k: (k, i, 0)),
        out_shape=jax.ShapeDtypeStruct((KVOL, R, C), jnp.float32),
    )(x, w)


def _make_scatter(second):
    mesh = plsc.VectorSubcoreMesh(core_axis_name="c", subcore_axis_name="s")

    @functools.partial(
        pl.kernel, mesh=mesh,
        compiler_params=pltpu.CompilerParams(use_tc_tiling_on_sc=False),
        out_type=jax.ShapeDtypeStruct((NP, C), jnp.float32),
        scratch_types=[
            pltpu.VMEM_SHARED((ACC_ROWS, Q), jnp.float32),
            pltpu.VMEM((SB, BLK), jnp.int32),
            pltpu.VMEM((SB, BLK), jnp.int32),
            pltpu.VMEM((4, BLK, Q), jnp.float32),
            pltpu.VMEM((ECH, Q), jnp.float32),
            pltpu.VMEM((ECH, Q), jnp.float32),
            pltpu.VMEM((Q,), jnp.float32),
            pltpu.VMEM((Q,), jnp.float32),
            pltpu.VMEM((16,), jnp.float32),
            pltpu.SemaphoreType.DMA,
            pltpu.SemaphoreType.DMA,
            pltpu.SemaphoreType.DMA,
            pltpu.SemaphoreType.DMA,
            pltpu.SemaphoreType.DMA,
            pltpu.SemaphoreType.DMA,
            pltpu.SemaphoreType.DMA,
            pltpu.SemaphoreType.DMA,
        ])
    def k(y4_hbm, gidx_hbm, oidx_hbm, b_hbm, rq_hbm, slope_hbm, x_hbm,
          out_hbm, acc, gidx_v, oidx_v, msg_v, ebuf, xbuf, b_v, rq_v,
          slope_v, sem_g0, sem_g1, sem_g2, sem_g3, sem_s0, sem_s1,
          sem_s2, sem_s3):
        c = lax.axis_index("c")
        s = lax.axis_index("s")
        sem_g = [sem_g0, sem_g1, sem_g2, sem_g3]
        sem_s = [sem_s0, sem_s1, sem_s2, sem_s3]
        NS = 4
        zeros16 = jnp.zeros((16,), jnp.float32)

        def do_chunk(q, cc):
            pltpu.sync_copy(gidx_hbm.at[q, s, pl.ds(cc * SB, SB)], gidx_v)
            pltpu.sync_copy(oidx_hbm.at[s, pl.ds(cc * SB, SB)], oidx_v)
            gv, ov = gidx_v, oidx_v
            for t in range(NS - 1):
                pltpu.async_copy(y4_hbm.at[gv.at[t]], msg_v.at[t],
                                 sem_g[t])
            for j in range(SB):
                a = j % NS
                pltpu.make_async_copy(y4_hbm.at[gv.at[j]], msg_v.at[a],
                                      sem_g[a]).wait()
                pltpu.async_copy(msg_v.at[a], acc.at[ov.at[j]], sem_s[a],
                                 add=True)
                if j + NS - 1 < SB:
                    na = (j + NS - 1) % NS
                    if j >= 1:
                        pltpu.make_async_copy(
                            msg_v.at[na], acc.at[ov.at[j - 1]],
                            sem_s[na]).wait()
                    pltpu.async_copy(y4_hbm.at[gv.at[j + NS - 1]],
                                     msg_v.at[na], sem_g[na])
            for j in range(SB - NS, SB):
                pltpu.make_async_copy(msg_v.at[j % NS],
                                      acc.at[ov.at[j]],
                                      sem_s[j % NS]).wait()

        for p in range(2):
            q = 2 * c + p

            def zbody(i, _):
                ebuf[i, pl.ds(0, 16)] = zeros16
                ebuf[i, pl.ds(16, 16)] = zeros16
                return 0
            lax.fori_loop(0, ECH, zbody, 0)

            def zcopy(z, _):
                r0 = pl.multiple_of(s * RPT + z * ECH, 8)
                pltpu.sync_copy(ebuf, acc.at[pl.ds(r0, ECH)])
                return 0
            lax.fori_loop(0, NCH, zcopy, 0)
            plsc.subcore_barrier()

            def cbody(cc, _):
                do_chunk(q, cc)
                return 0
            lax.fori_loop(0, NCHUNK, cbody, 0)
            plsc.subcore_barrier()

            qoff = pl.multiple_of(q * Q, Q)
            pltpu.sync_copy(b_hbm.at[pl.ds(qoff, Q)], b_v)
            pltpu.sync_copy(rq_hbm.at[pl.ds(qoff, Q)], rq_v)
            pltpu.sync_copy(slope_hbm, slope_v)
            slope = slope_v[...]
            for e in range(NCH):
                r0 = pl.multiple_of(s * RPT + e * ECH, 8)
                pltpu.sync_copy(acc.at[pl.ds(r0, ECH)], ebuf)
                if second:
                    pltpu.sync_copy(
                        x_hbm.at[pl.ds(r0, ECH), pl.ds(qoff, Q)], xbuf)

                def ebody(rr, _):
                    for dr in range(2):
                        r = 2 * rr + dr
                        for hh in range(2):
                            sl = pl.ds(hh * 16, 16)
                            v = ebuf[r, sl] + b_v[sl]
                            if second:
                                v = v * rq_v[sl] + xbuf[r, sl]
                                v = jnp.where(v >= 0, v, v * slope)
                            else:
                                v = jnp.where(v >= 0, v, v * slope)
                                v = v * rq_v[sl]
                            ebuf[r, sl] = v
                    return 0
                lax.fori_loop(0, ECH // 2, ebody, 0)
                pltpu.sync_copy(
                    ebuf, out_hbm.at[pl.ds(r0, ECH), pl.ds(qoff, Q)])
            plsc.subcore_barrier()

    return k


_scatter1 = _make_scatter(second=False)
_scatter2 = _make_scatter(second=True)


def kernel(in_feats, w1, b1, slope1, rq1, w2, b2, rq2, slope2, in_maps,
           out_maps):
    x = in_feats

    def mk_g4(rows):
        koff = jnp.arange(KVOL, dtype=jnp.int32)[:, None] * rows
        g = ((koff + in_maps) * 4).reshape(-1)
        g = jnp.concatenate([g, jnp.zeros((EPAD - NE,), jnp.int32)])
        return (g[None, :] + jnp.arange(4, dtype=jnp.int32)[:, None]
                ).reshape(4, NT, NBLK, BLK)

    g4a = mk_g4(N)
    g4b = mk_g4(NP)
    o = jnp.concatenate(
        [out_maps.reshape(-1),
         jnp.full((EPAD - NE,), DUMMY, jnp.int32)]).reshape(NT, NBLK, BLK)
    s1 = jnp.broadcast_to(slope1, (16,))
    s2 = jnp.broadcast_to(slope2, (16,))
    xp = jnp.pad(x, ((0, NP - N), (0, 0)))

    y1 = _gemm(x, w1, 2000).reshape(KVOL * N * 4, Q)
    hp = _scatter1(y1, g4a, o, b1, rq1, s1, xp)
    y2 = _gemm(hp, w2, 6256).reshape(KVOL * NP * 4, Q)
    return _scatter2(y2, g4b, o, b2, rq2, s2, xp)[:N]


# speedup vs baseline: 2.0157x; 1.1269x over previous
import functools

import jax
import jax.numpy as jnp
from jax import lax
from jax.experimental import pallas as pl
from jax.experimental.pallas import tpu as pltpu
from jax.experimental.pallas import tpu_sc as plsc

N = 50000
C = 128
KVOL = 27
E = 23000
NT = 16
NE = KVOL * E
BLK = 128
NBLK = 304
EPAD = NT * NBLK * BLK
Q = 32
NP = 50048
RPT = NP // NT
ECH = 136
NCH = RPT // ECH
DUMMY = NP
ACC_ROWS = NP + 8
SB = 19
NCHUNK = NBLK // SB


def _gemm(x, w, BN):
    R = x.shape[0]

    def body(x_ref, w_ref, y_ref):
        y_ref[...] = lax.dot_general(
            x_ref[...], w_ref[0],
            (((1,), (1,)), ((), ())),
            preferred_element_type=jnp.float32)[None]

    return pl.pallas_call(
        body,
        grid=(R // BN, KVOL),
        in_specs=[
            pl.BlockSpec((BN, C), lambda i, k: (i, 0)),
            pl.BlockSpec((1, C, C), lambda i, k: (k, 0, 0)),
        ],
        out_specs=pl.BlockSpec((1, BN, C), lambda i, k: (k, i, 0)),
        out_shape=jax.ShapeDtypeStruct((KVOL, R, C), jnp.float32),
    )(x, w)


def _make_scatter(second):
    mesh = plsc.VectorSubcoreMesh(core_axis_name="c", subcore_axis_name="s")

    @functools.partial(
        pl.kernel, mesh=mesh,
        compiler_params=pltpu.CompilerParams(use_tc_tiling_on_sc=False),
        out_type=jax.ShapeDtypeStruct((NP, C), jnp.float32),
        scratch_types=[
            pltpu.VMEM_SHARED((ACC_ROWS, Q), jnp.float32),
            pltpu.VMEM((SB, BLK), jnp.int32),
            pltpu.VMEM((SB, BLK), jnp.int32),
            pltpu.VMEM((4, BLK, Q), jnp.float32),
            pltpu.VMEM((ECH, Q), jnp.float32),
            pltpu.VMEM((ECH, Q), jnp.float32),
            pltpu.VMEM((Q,), jnp.float32),
            pltpu.VMEM((Q,), jnp.float32),
            pltpu.VMEM((16,), jnp.float32),
            pltpu.SemaphoreType.DMA,
            pltpu.SemaphoreType.DMA,
            pltpu.SemaphoreType.DMA,
            pltpu.SemaphoreType.DMA,
            pltpu.SemaphoreType.DMA,
            pltpu.SemaphoreType.DMA,
            pltpu.SemaphoreType.DMA,
            pltpu.SemaphoreType.DMA,
        ])
    def k(y4_hbm, gidx_hbm, oidx_hbm, b_hbm, rq_hbm, slope_hbm, x_hbm,
          out_hbm, acc, gidx_v, oidx_v, msg_v, ebuf, xbuf, b_v, rq_v,
          slope_v, sem_g0, sem_g1, sem_g2, sem_g3, sem_s0, sem_s1,
          sem_s2, sem_s3):
        c = lax.axis_index("c")
        s = lax.axis_index("s")
        sem_g = [sem_g0, sem_g1, sem_g2, sem_g3]
        sem_s = [sem_s0, sem_s1, sem_s2, sem_s3]
        NS = 4
        zeros16 = jnp.zeros((16,), jnp.float32)

        def do_chunk(q, cc):
            pltpu.sync_copy(gidx_hbm.at[q, s, pl.ds(cc * SB, SB)], gidx_v)
            pltpu.sync_copy(oidx_hbm.at[s, pl.ds(cc * SB, SB)], oidx_v)
            gv, ov = gidx_v, oidx_v
            for t in range(NS - 1):
                pltpu.async_copy(y4_hbm.at[gv.at[t]], msg_v.at[t],
                                 sem_g[t])
            for j in range(SB):
                a = j % NS
                pltpu.make_async_copy(y4_hbm.at[gv.at[j]], msg_v.at[a],
                                      sem_g[a]).wait()
                pltpu.async_copy(msg_v.at[a], acc.at[ov.at[j]], sem_s[a],
                                 add=True)
                if j + NS - 1 < SB:
                    na = (j + NS - 1) % NS
                    if j >= 1:
                        pltpu.make_async_copy(
                            msg_v.at[na], acc.at[ov.at[j - 1]],
                            sem_s[na]).wait()
                    pltpu.async_copy(y4_hbm.at[gv.at[j + NS - 1]],
                                     msg_v.at[na], sem_g[na])
            for j in range(SB - NS, SB):
                pltpu.make_async_copy(msg_v.at[j % NS],
                                      acc.at[ov.at[j]],
                                      sem_s[j % NS]).wait()

        for p in range(2):
            q = 2 * c + p

            def zbody(i, _):
                ebuf[i, pl.ds(0, 16)] = zeros16
                ebuf[i, pl.ds(16, 16)] = zeros16
                return 0
            lax.fori_loop(0, ECH, zbody, 0)

            def zcopy(z, _):
                r0 = pl.multiple_of(s * RPT + z * ECH, 8)
                pltpu.sync_copy(ebuf, acc.at[pl.ds(r0, ECH)])
                return 0
            lax.fori_loop(0, NCH, zcopy, 0)
            plsc.subcore_barrier()

            def cbody(cc, _):
                do_chunk(q, cc)
                return 0
            lax.fori_loop(0, NCHUNK, cbody, 0)
            plsc.subcore_barrier()

            qoff = pl.multiple_of(q * Q, Q)
            pltpu.sync_copy(b_hbm.at[pl.ds(qoff, Q)], b_v)
            pltpu.sync_copy(rq_hbm.at[pl.ds(qoff, Q)], rq_v)
            pltpu.sync_copy(slope_hbm, slope_v)
            slope = slope_v[...]
            for e in range(NCH):
                r0 = pl.multiple_of(s * RPT + e * ECH, 8)
                pltpu.sync_copy(acc.at[pl.ds(r0, ECH)], ebuf)
                if second:
                    pltpu.sync_copy(
                        x_hbm.at[pl.ds(r0, ECH), pl.ds(qoff, Q)], xbuf)

                def ebody(rr, _):
                    for dr in range(2):
                        r = 2 * rr + dr
                        for hh in range(2):
                            sl = pl.ds(hh * 16, 16)
                            v = ebuf[r, sl] + b_v[sl]
                            if second:
                                v = v * rq_v[sl] + xbuf[r, sl]
                                v = jnp.where(v >= 0, v, v * slope)
                            else:
                                v = jnp.where(v >= 0, v, v * slope)
                                v = v * rq_v[sl]
                            ebuf[r, sl] = v
                    return 0
                lax.fori_loop(0, ECH // 2, ebody, 0)
                pltpu.sync_copy(
                    ebuf, out_hbm.at[pl.ds(r0, ECH), pl.ds(qoff, Q)])
            plsc.subcore_barrier()

    return k


_scatter1 = _make_scatter(second=False)
_scatter2 = _make_scatter(second=True)


def kernel(in_feats, w1, b1, slope1, rq1, w2, b2, rq2, slope2, in_maps,
           out_maps):
    x = in_feats

    def mk_g4(rows):
        koff = jnp.arange(KVOL, dtype=jnp.int32)[:, None] * rows
        g = ((koff + in_maps) * 4).reshape(-1)
        g = jnp.concatenate([g, jnp.zeros((EPAD - NE,), jnp.int32)])
        return (g[None, :] + jnp.arange(4, dtype=jnp.int32)[:, None]
                ).reshape(4, NT, NBLK, BLK)

    g4a = mk_g4(N)
    g4b = mk_g4(NP)
    o = jnp.concatenate(
        [out_maps.reshape(-1),
         jnp.full((EPAD - NE,), DUMMY, jnp.int32)]).reshape(NT, NBLK, BLK)
    s1 = jnp.broadcast_to(slope1, (16,))
    s2 = jnp.broadcast_to(slope2, (16,))
    xp = jnp.pad(x, ((0, NP - N), (0, 0)))

    y1 = _gemm(x, w1, 5000).reshape(KVOL * N * 4, Q)
    hp = _scatter1(y1, g4a, o, b1, rq1, s1, xp)
    y2 = _gemm(hp, w2, 6256).reshape(KVOL * NP * 4, Q)
    return _scatter2(y2, g4b, o, b2, rq2, s2, xp)[:N]
